# baseline (speedup 1.0000x reference)
import functools

import jax
import jax.numpy as jnp
from jax import lax
from jax.experimental import pallas as pl
from jax.experimental.pallas import tpu as pltpu
from jax.experimental.pallas import tpu_sc as plsc


def kernel(seq_len, relative_embeddings):
    del seq_len
    two_max_len, embed = relative_embeddings.shape
    s = two_max_len // 2
    row_words = s * embed
    table_words = two_max_len * embed

    info = plsc.get_sparse_core_info()
    num_workers = info.num_cores * info.num_subcores
    rows_per_w = s // num_workers

    mesh = plsc.VectorSubcoreMesh(core_axis_name="c", subcore_axis_name="s")

    @functools.partial(
        pl.kernel,
        mesh=mesh,
        out_type=jax.ShapeDtypeStruct((s * row_words,), jnp.float32),
        scratch_types=[
            pltpu.VMEM((table_words,), jnp.float32),
            pltpu.SemaphoreType.DMA,
        ],
    )
    def toeplitz_rows(table_hbm, out_hbm, table_v, sem):
        wid = lax.axis_index("s") * info.num_cores + lax.axis_index("c")
        pltpu.sync_copy(table_hbm, table_v)
        base = wid * rows_per_w

        def fire(r, carry):
            i = base + r
            src_start = (s - i) * embed
            pltpu.async_copy(
                table_v.at[pl.ds(src_start, row_words)],
                out_hbm.at[pl.ds(i * row_words, row_words)],
                sem,
            )
            return carry

        lax.fori_loop(0, rows_per_w, fire, 0)

        def drain(r, carry):
            i = base + r
            pltpu.make_async_copy(
                table_v.at[pl.ds(0, row_words)],
                out_hbm.at[pl.ds(i * row_words, row_words)],
                sem,
            ).wait()
            return carry

        lax.fori_loop(0, rows_per_w, drain, 0)

    flat = toeplitz_rows(relative_embeddings.reshape(table_words))
    return flat.reshape(s, s, embed)


# speedup vs baseline: 3.0799x; 3.0799x over previous
import functools

import jax
import jax.numpy as jnp
from jax import lax
from jax.experimental import pallas as pl
from jax.experimental.pallas import tpu as pltpu
from jax.experimental.pallas import tpu_sc as plsc

_LANE = 128


def kernel(seq_len, relative_embeddings):
    del seq_len
    two_max_len, embed = relative_embeddings.shape
    s = two_max_len // 2
    row_blocks = s * embed // _LANE
    g_rows = two_max_len * embed // _LANE + 2

    info = plsc.get_sparse_core_info()
    num_workers = info.num_cores * info.num_subcores
    tiles_per_class = num_workers // 8
    rows_per_w = s // num_workers

    flat = relative_embeddings.reshape(-1)
    padded = jnp.pad(flat, (0, g_rows * _LANE + 112 - flat.shape[0]))
    shifted = jnp.stack(
        [lax.dynamic_slice(padded, (o,), (g_rows * _LANE,))
         for o in range(0, 128, 16)]
    ).reshape(8, g_rows, _LANE)

    mesh = plsc.VectorSubcoreMesh(core_axis_name="c", subcore_axis_name="s")

    @functools.partial(
        pl.kernel,
        mesh=mesh,
        out_type=jax.ShapeDtypeStruct((s, row_blocks, _LANE), jnp.float32),
        scratch_types=[
            pltpu.VMEM((1, g_rows, _LANE), jnp.float32),
            pltpu.SemaphoreType.DMA,
        ],
    )
    def toeplitz_rows(shifted_hbm, out_hbm, table_v, sem):
        wid = lax.axis_index("s") * info.num_cores + lax.axis_index("c")
        r = wid // tiles_per_class
        q0 = (wid % tiles_per_class) * rows_per_w
        o_idx = (8 - r) % 8
        base_m = jnp.where(r == 0, 256, 255)
        pltpu.sync_copy(shifted_hbm.at[pl.ds(o_idx, 1)], table_v)

        def fire(local, carry):
            q = q0 + local
            i = 8 * q + r
            m = base_m - q
            pltpu.async_copy(
                table_v.at[:, pl.ds(m, row_blocks), :],
                out_hbm.at[pl.ds(i, 1)],
                sem,
            )
            return carry

        lax.fori_loop(0, rows_per_w, fire, 0)

        def drain(local, carry):
            i = 8 * (q0 + local) + r
            pltpu.make_async_copy(
                table_v.at[:, pl.ds(0, row_blocks), :],
                out_hbm.at[pl.ds(i, 1)],
                sem,
            ).wait()
            return carry

        lax.fori_loop(0, rows_per_w, drain, 0)

    out = toeplitz_rows(shifted)
    return out.reshape(s, s, embed)
